# Initial kernel scaffold; baseline (speedup 1.0000x reference)
import jax
import jax.numpy as jnp
from jax.experimental import pallas as pl


def kernel(waves_re, waves_im, x0_0, y0_0, x0_1, y0_1, x0_2, y0_2, x_out, y_out):
    raise NotImplementedError("write your pallas kernel here")


# speedup vs baseline: 1.3847x; 1.3847x over previous
import cmath
import functools
import math

import jax
import jax.numpy as jnp
from jax import lax
from jax.experimental import pallas as pl
from jax.experimental.pallas import tpu as pltpu

_LAMBDA0 = 1.55e-6
_LAMBDA = _LAMBDA0 / 1.45
_W0 = 5.0e-7
_K_RSM = 0.6
_K_GBM = 0.4
_K_SUB = 2.0 * math.pi * 1.45 / _LAMBDA0
_DELTA = 1.0e-7
_H_NEURON = 3e-6
_TM02_BETA = 2.0 * math.pi * 2.0 / _LAMBDA0

_C = (0.8 * cmath.exp(-1j * _TM02_BETA * _H_NEURON / 2.0)
      * 0.3 * cmath.exp(1j * 0.5))
_C_RE = float(_C.real)
_C_IM = float(_C.imag)

_C_RSM = float(_K_RSM * math.sqrt(2.0 * _W0 / math.sqrt(math.pi)))
_C_WZ = float(_LAMBDA / (math.pi * _W0 * _W0))
_C_INV_W0SQ = float(1.0 / (_W0 * _W0))
_K_SUB_OVER_2PI = float(_K_SUB / (2.0 * math.pi))
_NEG_TWO_PI = float(-2.0 * math.pi)


def _transfer_kernel(x0_ref, y0s_ref, xc1_ref, yc1_ref, xc2_ref, yc2_ref,
                     xc3_ref, yc3_ref, u_ref, v_ref, *, rows_per_block,
                     n0, n01):
    row0 = pl.program_id(0) * rows_per_block
    rows = row0 + lax.broadcasted_iota(jnp.int32, (rows_per_block, 1), 0)
    xc = jnp.where(rows < n0, xc1_ref[...],
                   jnp.where(rows < n01, xc2_ref[...], xc3_ref[...]))
    yc = jnp.where(rows < n0, yc1_ref[...],
                   jnp.where(rows < n01, yc2_ref[...], yc3_ref[...]))
    x0 = x0_ref[...]
    y0s = y0s_ref[...]

    r0 = xc - x0
    z = jnp.abs(yc - y0s)
    r0sq = r0 * r0
    rsq = r0sq + z * z
    inv_r = lax.rsqrt(rsq)
    r = rsq * inv_r
    e_rsm = _C_RSM * jnp.sqrt(inv_r) * (z * inv_r)
    wz = z * _C_WZ
    w0w = lax.rsqrt(1.0 + wz * wz)
    e_gbm = _K_GBM * jnp.sqrt(w0w) * jnp.exp(-r0sq * (w0w * w0w) * _C_INV_W0SQ)
    amp = e_rsm + e_gbm
    turns = r * _K_SUB_OVER_2PI
    frac = turns - jnp.round(turns)
    theta = frac * _NEG_TWO_PI
    u = amp * jnp.cos(theta)
    v = amp * jnp.sin(theta)
    u_ref[...] = (_C_RE * u - _C_IM * v).astype(jnp.bfloat16)
    v_ref[...] = (_C_RE * v + _C_IM * u).astype(jnp.bfloat16)


def _network_kernel(wre_ref, wim_ref, u_ref, v_ref, o_ref, *, offs):
    re = wre_ref[...].astype(jnp.bfloat16)
    im = wim_ref[...].astype(jnp.bfloat16)
    new_re = new_im = None
    for li, off in enumerate(offs):
        k = re.shape[1]
        u = u_ref[off:off + k, :]
        v = v_ref[off:off + k, :]
        new_re = (jnp.dot(re, u, preferred_element_type=jnp.float32)
                  - jnp.dot(im, v, preferred_element_type=jnp.float32))
        new_im = (jnp.dot(re, v, preferred_element_type=jnp.float32)
                  + jnp.dot(im, u, preferred_element_type=jnp.float32))
        if li + 1 < len(offs):
            re = new_re.astype(jnp.bfloat16)
            im = new_im.astype(jnp.bfloat16)
    mag = jnp.sqrt(new_re * new_re + new_im * new_im)
    m = jnp.max(mag, axis=-1, keepdims=True)
    e = jnp.exp(mag - m)
    s = jnp.sum(e, axis=-1, keepdims=True)
    o_ref[...] = e * pl.reciprocal(s, approx=True)


def kernel(waves_re, waves_im, x0_0, y0_0, x0_1, y0_1, x0_2, y0_2,
           x_out, y_out):
    f32 = jnp.float32
    B, N0 = waves_re.shape
    N1, N2, L = int(x0_1.size), int(x0_2.size), int(x_out.size)
    NT = N0 + N1 + N2

    x0_all = jnp.concatenate([x0_0, x0_1, x0_2]).reshape(NT, 1).astype(f32)
    y0s_all = (jnp.concatenate([y0_0, y0_1, y0_2]).reshape(NT, 1)
               - _H_NEURON - _DELTA).astype(f32)
    xc1 = x0_1.reshape(1, N1).astype(f32)
    yc1 = y0_1.reshape(1, N1).astype(f32)
    xc2 = x0_2.reshape(1, N2).astype(f32)
    yc2 = y0_2.reshape(1, N2).astype(f32)
    xc3 = x_out.reshape(1, L).astype(f32)
    yc3 = y_out.reshape(1, L).astype(f32)

    na = 2 if (NT % 2 == 0 and (NT // 2) % 16 == 0) else 1
    rows = NT // na
    u_all, v_all = pl.pallas_call(
        functools.partial(_transfer_kernel, rows_per_block=rows,
                          n0=N0, n01=N0 + N1),
        out_shape=(jax.ShapeDtypeStruct((NT, L), jnp.bfloat16),
                   jax.ShapeDtypeStruct((NT, L), jnp.bfloat16)),
        grid=(na,),
        in_specs=[
            pl.BlockSpec((rows, 1), lambda i: (i, 0)),
            pl.BlockSpec((rows, 1), lambda i: (i, 0)),
            pl.BlockSpec((1, N1), lambda i: (0, 0)),
            pl.BlockSpec((1, N1), lambda i: (0, 0)),
            pl.BlockSpec((1, N2), lambda i: (0, 0)),
            pl.BlockSpec((1, N2), lambda i: (0, 0)),
            pl.BlockSpec((1, L), lambda i: (0, 0)),
            pl.BlockSpec((1, L), lambda i: (0, 0)),
        ],
        out_specs=[pl.BlockSpec((rows, L), lambda i: (i, 0)),
                   pl.BlockSpec((rows, L), lambda i: (i, 0))],
        compiler_params=pltpu.CompilerParams(
            dimension_semantics=("parallel",)),
    )(x0_all, y0s_all, xc1, yc1, xc2, yc2, xc3, yc3)

    BT = 256 if B % 256 == 0 else B
    return pl.pallas_call(
        functools.partial(_network_kernel, offs=(0, N0, N0 + N1)),
        out_shape=jax.ShapeDtypeStruct((B, L), f32),
        grid=(B // BT,),
        in_specs=[
            pl.BlockSpec((BT, N0), lambda i: (i, 0)),
            pl.BlockSpec((BT, N0), lambda i: (i, 0)),
            pl.BlockSpec((NT, L), lambda i: (0, 0)),
            pl.BlockSpec((NT, L), lambda i: (0, 0)),
        ],
        out_specs=pl.BlockSpec((BT, L), lambda i: (i, 0)),
        compiler_params=pltpu.CompilerParams(
            dimension_semantics=("parallel",)),
    )(waves_re, waves_im, u_all, v_all)
